# baseline (speedup 1.0000x reference)
import functools

import jax
import jax.numpy as jnp
from jax import lax
from jax.experimental import pallas as pl
from jax.experimental.pallas import tpu as pltpu
from jax.experimental.pallas import tpu_sc as plsc

F = 26
V = 100000
K = 16
B = 4096
H1, H2 = 400, 400
D0 = F * K

NC, NS = 2, 16
NW = NC * NS
BPW = B // NW


PPW = D0 // NW


def _sc_gather_body(idxT_hbm, fidxT_hbm, embT_hbm, lin_hbm, xT_out, linT_out,
                    plane_v, idx_v, row_v, fidx_v, linT_v, sem_p, sem_l):
    wid = lax.axis_index("s") * NC + lax.axis_index("c")

    b0 = pl.multiple_of(wid * BPW, BPW)
    pltpu.sync_copy(fidxT_hbm.at[:, pl.ds(b0, BPW)], fidx_v)

    def lin_fire(f, carry):
        pltpu.async_copy(lin_hbm.at[fidx_v.at[f]], linT_v.at[f], sem_l)
        return carry

    lax.fori_loop(0, F, lin_fire, 0)

    def plane(i, carry):
        p = wid * PPW + i
        f = p // K
        k = p % K
        pltpu.sync_copy(embT_hbm.at[f, k], plane_v)
        pltpu.sync_copy(idxT_hbm.at[f], idx_v)

        def grp(g, c2):
            off = pl.multiple_of(g * 16, 16)
            vvec = idx_v[pl.ds(off, 16)]
            row_v[pl.ds(off, 16)] = plsc.load_gather(plane_v, [vvec])
            return c2

        lax.fori_loop(0, B // 16, grp, 0)
        pltpu.sync_copy(row_v, xT_out.at[p])
        return carry

    lax.fori_loop(0, PPW, plane, 0)

    def lin_drain(f, carry):
        pltpu.make_async_copy(lin_hbm.at[fidx_v.at[f]],
                              linT_v.at[f], sem_l).wait()
        return carry

    lax.fori_loop(0, F, lin_drain, 0)
    pltpu.sync_copy(linT_v, linT_out.at[:, pl.ds(b0, BPW)])


_sc_gather = functools.partial(
    pl.kernel,
    mesh=plsc.VectorSubcoreMesh(core_axis_name="c", subcore_axis_name="s",
                                num_cores=NC, num_subcores=NS),
    compiler_params=pltpu.CompilerParams(needs_layout_passes=False),
    out_type=[
        jax.ShapeDtypeStruct((D0, B), jnp.float32),
        jax.ShapeDtypeStruct((F, B), jnp.float32),
    ],
    scratch_types=[
        pltpu.VMEM((V,), jnp.float32),
        pltpu.VMEM((B,), jnp.int32),
        pltpu.VMEM((B,), jnp.float32),
        pltpu.VMEM((F, BPW), jnp.int32),
        pltpu.VMEM((F, BPW), jnp.float32),
        pltpu.SemaphoreType.DMA,
        pltpu.SemaphoreType.DMA,
    ],
)(_sc_gather_body)


BB = 4096


def _mlp_body(xt_ref, lin_ref, m_ref, w1t_ref, b1_ref, w2t_ref, b2_ref,
              w3t_ref, b3_ref, lb_ref, out_ref, inter_ref, acc_ref):
    i = pl.program_id(0)
    xt = xt_ref[...]
    st = jnp.dot(m_ref[...], xt, precision=lax.Precision.HIGHEST)
    part = 0.5 * (jnp.sum(st * st, axis=(0, 1), keepdims=True)
                  - jnp.sum(xt * xt, axis=(0, 1), keepdims=True))

    @pl.when(i == 0)
    def _():
        acc_ref[...] = jnp.zeros((1, 1), jnp.float32)

    acc_ref[...] += part
    h = jnp.maximum(
        jnp.dot(w1t_ref[...], xt, precision=lax.Precision.DEFAULT)
        + b1_ref[...], 0.0)
    h = jnp.maximum(
        jnp.dot(w2t_ref[...], h, precision=lax.Precision.DEFAULT)
        + b2_ref[...], 0.0)
    fnn = jnp.dot(w3t_ref[...], h, precision=lax.Precision.DEFAULT) + b3_ref[...]
    line = jnp.sum(lin_ref[...], axis=0, keepdims=True) + lb_ref[...]
    out_ref[...] = line + fnn
    inter_ref[...] = acc_ref[...]


_mlp = pl.pallas_call(
    _mlp_body,
    grid=(B // BB,),
    in_specs=[
        pl.BlockSpec((D0, BB), lambda i: (0, i)),
        pl.BlockSpec((F, BB), lambda i: (0, i)),
        pl.BlockSpec((K, D0), lambda i: (0, 0)),
        pl.BlockSpec((H1, D0), lambda i: (0, 0)),
        pl.BlockSpec((H1, 1), lambda i: (0, 0)),
        pl.BlockSpec((H2, H1), lambda i: (0, 0)),
        pl.BlockSpec((H2, 1), lambda i: (0, 0)),
        pl.BlockSpec((1, H2), lambda i: (0, 0)),
        pl.BlockSpec((1, 1), lambda i: (0, 0)),
        pl.BlockSpec((1, 1), lambda i: (0, 0)),
    ],
    out_specs=[
        pl.BlockSpec((1, BB), lambda i: (0, i)),
        pl.BlockSpec((1, 1), lambda i: (0, 0)),
    ],
    out_shape=[
        jax.ShapeDtypeStruct((1, B), jnp.float32),
        jax.ShapeDtypeStruct((1, 1), jnp.float32),
    ],
    scratch_shapes=[pltpu.VMEM((1, 1), jnp.float32)],
)


def kernel(inputs, emb_table, lin_table, lin_bias, W1, b1, W2, b2, W3, b3):
    idxT = inputs.T
    fidxT = idxT + (jnp.arange(F, dtype=jnp.int32) * V)[:, None]
    embT = jnp.transpose(emb_table, (0, 2, 1))
    lin_flat = lin_table.reshape(F * V)
    xT, linT = _sc_gather(idxT, fidxT, embT, lin_flat)
    msel = jnp.tile(jnp.eye(K, dtype=jnp.float32), (1, F))
    outT, inter = _mlp(xT, linT, msel, W1.T, b1.reshape(H1, 1), W2.T,
                       b2.reshape(H2, 1), W3.T, b3.reshape(1, 1),
                       lin_bias.reshape(1, 1))
    return outT.reshape(B, 1) + inter


# speedup vs baseline: 1.0007x; 1.0007x over previous
import functools

import jax
import jax.numpy as jnp
from jax import lax
from jax.experimental import pallas as pl
from jax.experimental.pallas import tpu as pltpu
from jax.experimental.pallas import tpu_sc as plsc

F = 26
V = 100000
K = 16
B = 4096
H1, H2 = 400, 400
D0 = F * K

NC, NS = 2, 16
NW = NC * NS
BPW = B // NW


PPW = D0 // NW


def _sc_gather_body(idxT_hbm, fidxT_hbm, embT_hbm, lin_hbm, xT_out, linT_out,
                    plane_v, idx_v, row_v, fidx_v, linT_v, sem_p, sem_l):
    wid = lax.axis_index("s") * NC + lax.axis_index("c")

    b0 = pl.multiple_of(wid * BPW, BPW)
    pltpu.sync_copy(fidxT_hbm.at[:, pl.ds(b0, BPW)], fidx_v)

    def lin_fire(f, carry):
        pltpu.async_copy(lin_hbm.at[fidx_v.at[f]], linT_v.at[f], sem_l)
        return carry

    lax.fori_loop(0, F, lin_fire, 0)

    def plane(i, carry):
        p = wid * PPW + i
        f = p // K
        k = p % K
        pltpu.sync_copy(embT_hbm.at[f, k], plane_v)
        pltpu.sync_copy(idxT_hbm.at[f], idx_v)

        def grp(g, c2):
            off = pl.multiple_of(g * 16, 16)
            vvec = idx_v[pl.ds(off, 16)]
            row_v[pl.ds(off, 16)] = plsc.load_gather(plane_v, [vvec])
            return c2

        lax.fori_loop(0, B // 16, grp, 0)
        pltpu.sync_copy(row_v, xT_out.at[p])
        return carry

    lax.fori_loop(0, PPW, plane, 0)

    def lin_drain(f, carry):
        pltpu.make_async_copy(lin_hbm.at[fidx_v.at[f]],
                              linT_v.at[f], sem_l).wait()
        return carry

    lax.fori_loop(0, F, lin_drain, 0)
    pltpu.sync_copy(linT_v, linT_out.at[:, pl.ds(b0, BPW)])


_sc_gather = functools.partial(
    pl.kernel,
    mesh=plsc.VectorSubcoreMesh(core_axis_name="c", subcore_axis_name="s",
                                num_cores=NC, num_subcores=NS),
    compiler_params=pltpu.CompilerParams(needs_layout_passes=False),
    out_type=[
        jax.ShapeDtypeStruct((D0, B), jnp.float32),
        jax.ShapeDtypeStruct((F, B), jnp.float32),
    ],
    scratch_types=[
        pltpu.VMEM((V,), jnp.float32),
        pltpu.VMEM((B,), jnp.int32),
        pltpu.VMEM((B,), jnp.float32),
        pltpu.VMEM((F, BPW), jnp.int32),
        pltpu.VMEM((F, BPW), jnp.float32),
        pltpu.SemaphoreType.DMA,
        pltpu.SemaphoreType.DMA,
    ],
)(_sc_gather_body)


BB = 4096


def _mlp_body(xt_ref, lin_ref, m_ref, w1t_ref, b1_ref, w2t_ref, b2_ref,
              w3t_ref, b3_ref, lb_ref, out_ref, inter_ref, acc_ref):
    i = pl.program_id(0)
    xt = xt_ref[...]
    st = jnp.dot(m_ref[...], xt, precision=lax.Precision.HIGHEST)
    part = 0.5 * (jnp.sum(st * st, axis=(0, 1), keepdims=True)
                  - jnp.sum(xt * xt, axis=(0, 1), keepdims=True))

    @pl.when(i == 0)
    def _():
        acc_ref[...] = jnp.zeros((1, 1), jnp.float32)

    acc_ref[...] += part
    xb = xt.astype(jnp.bfloat16)
    h = jnp.maximum(
        jnp.dot(w1t_ref[...].astype(jnp.bfloat16), xb,
                preferred_element_type=jnp.float32) + b1_ref[...], 0.0)
    h = jnp.maximum(
        jnp.dot(w2t_ref[...].astype(jnp.bfloat16), h.astype(jnp.bfloat16),
                preferred_element_type=jnp.float32) + b2_ref[...], 0.0)
    fnn = jnp.dot(w3t_ref[...].astype(jnp.bfloat16), h.astype(jnp.bfloat16),
                  preferred_element_type=jnp.float32) + b3_ref[...]
    line = jnp.sum(lin_ref[...], axis=0, keepdims=True) + lb_ref[...]
    out_ref[...] = line + fnn
    inter_ref[...] = acc_ref[...]


_mlp = pl.pallas_call(
    _mlp_body,
    grid=(B // BB,),
    in_specs=[
        pl.BlockSpec((D0, BB), lambda i: (0, i)),
        pl.BlockSpec((F, BB), lambda i: (0, i)),
        pl.BlockSpec((K, D0), lambda i: (0, 0)),
        pl.BlockSpec((H1, D0), lambda i: (0, 0)),
        pl.BlockSpec((H1, 1), lambda i: (0, 0)),
        pl.BlockSpec((H2, H1), lambda i: (0, 0)),
        pl.BlockSpec((H2, 1), lambda i: (0, 0)),
        pl.BlockSpec((1, H2), lambda i: (0, 0)),
        pl.BlockSpec((1, 1), lambda i: (0, 0)),
        pl.BlockSpec((1, 1), lambda i: (0, 0)),
    ],
    out_specs=[
        pl.BlockSpec((1, BB), lambda i: (0, i)),
        pl.BlockSpec((1, 1), lambda i: (0, 0)),
    ],
    out_shape=[
        jax.ShapeDtypeStruct((1, B), jnp.float32),
        jax.ShapeDtypeStruct((1, 1), jnp.float32),
    ],
    scratch_shapes=[pltpu.VMEM((1, 1), jnp.float32)],
)


def kernel(inputs, emb_table, lin_table, lin_bias, W1, b1, W2, b2, W3, b3):
    idxT = inputs.T
    fidxT = idxT + (jnp.arange(F, dtype=jnp.int32) * V)[:, None]
    embT = jnp.transpose(emb_table, (0, 2, 1))
    lin_flat = lin_table.reshape(F * V)
    xT, linT = _sc_gather(idxT, fidxT, embT, lin_flat)
    msel = jnp.tile(jnp.eye(K, dtype=jnp.float32), (1, F))
    outT, inter = _mlp(xT, linT, msel, W1.T, b1.reshape(H1, 1), W2.T,
                       b2.reshape(H2, 1), W3.T, b3.reshape(1, 1),
                       lin_bias.reshape(1, 1))
    return outT.reshape(B, 1) + inter
